# baseline (speedup 1.0000x reference)
import functools

import jax
import jax.numpy as jnp
from jax import lax
from jax.experimental import pallas as pl
from jax.experimental.pallas import tpu as pltpu
from jax.experimental.pallas import tpu_sc as plsc

TAU = 2.0
GAMMA = 0.7
EPS = 0.05
KS = 1000
KT = 100000
B = 1024

NUM_CORES = 2
NUM_SUBCORES = 16
NUM_WORKERS = NUM_CORES * NUM_SUBCORES
ROWS_PER_WORKER = B // NUM_WORKERS
LANES = 16
NVEC = (KS + LANES - 1) // LANES


_GRAN = 16


def _sc_gather(teacher, t_idx, s_idx):
    teacher3 = teacher.reshape(B, KT // _GRAN, _GRAN)
    mesh = plsc.VectorSubcoreMesh(
        core_axis_name="c", subcore_axis_name="s",
        num_cores=NUM_CORES, num_subcores=NUM_SUBCORES)

    @functools.partial(
        pl.kernel,
        out_type=jax.ShapeDtypeStruct((B, KS), jnp.float32),
        mesh=mesh,
        scratch_types=[
            pltpu.VMEM((KS,), jnp.int32),
            pltpu.VMEM((KS,), jnp.int32),
            pltpu.VMEM((KS,), jnp.int32),
            pltpu.VMEM((KS,), jnp.int32),
            pltpu.VMEM((KS, _GRAN), jnp.float32),
            pltpu.VMEM((KS,), jnp.float32),
            pltpu.SemaphoreType.DMA,
        ],
        compiler_params=pltpu.CompilerParams(
            needs_layout_passes=False, use_tc_tiling_on_sc=False),
    )
    def k(teacher_hbm, tidx_hbm, sidx_hbm, out_hbm, tv, sv, gv, lv, buf_v,
          row_v, sem):
        wid = lax.axis_index("s") * NUM_CORES + lax.axis_index("c")
        pltpu.sync_copy(tidx_hbm, tv)
        pltpu.sync_copy(sidx_hbm, sv)

        def comp(j, carry):
            off = jnp.minimum(j * LANES, KS - LANES)
            tvec = tv[pl.ds(off, LANES)]
            svec = sv[pl.ds(off, LANES)]
            plsc.store_scatter(gv, [svec], tvec >> 4)
            plsc.store_scatter(lv, [svec], tvec & 15)
            return carry
        lax.fori_loop(0, NVEC, comp, 0)

        base = wid * ROWS_PER_WORKER
        lane_iota = lax.iota(jnp.int32, LANES)

        def row(i, carry):
            b = base + i
            pltpu.async_copy(teacher_hbm.at[b].at[gv], buf_v, sem).wait()

            def extract(j, carry2):
                off = jnp.minimum(j * LANES, KS - LANES)
                rows = off + lane_iota
                lanes = lv[pl.ds(off, LANES)]
                row_v[pl.ds(off, LANES)] = plsc.load_gather(
                    buf_v, [rows, lanes])
                return carry2
            lax.fori_loop(0, NVEC, extract, 0)
            pltpu.sync_copy(row_v, out_hbm.at[b])
            return carry
        lax.fori_loop(0, ROWS_PER_WORKER, row, 0)

    return k(teacher3, t_idx, s_idx)


_BB = 128


def _tc_body(g_ref, s_ref, loss_ref, c_ref, w_ref):
    g = g_ref[...] * (1.0 / TAU)
    m = jnp.max(g, axis=1, keepdims=True)
    e = jnp.exp(g - m)
    p = e / jnp.sum(e, axis=1, keepdims=True)
    q = (1.0 - EPS) * p + (EPS / KS)

    st = s_ref[...] * (1.0 / TAU)
    ms = jnp.max(st, axis=1, keepdims=True)
    es = jnp.exp(st - ms)
    ps = es / jnp.sum(es, axis=1, keepdims=True)

    qc = jnp.maximum(q, 1e-12)
    psc = jnp.maximum(ps, 1e-12)
    kl = jnp.sum(qc * (jnp.log(qc) - jnp.log(psc)), axis=1)
    c = jnp.max(q, axis=1)
    w = jnp.clip((c - GAMMA) / (1.0 - GAMMA), 0.0, 1.0)
    c_ref[...] = c[:, None]
    w_ref[...] = w[:, None]

    part = jnp.sum(w * (TAU * TAU) * kl) * (1.0 / B)
    i = pl.program_id(0)

    @pl.when(i == 0)
    def _():
        loss_ref[...] = part[None, None]

    @pl.when(i != 0)
    def _():
        loss_ref[...] += part[None, None]


def _tc_loss(g, student, interpret=False):
    return pl.pallas_call(
        _tc_body,
        grid=(B // _BB,),
        in_specs=[
            pl.BlockSpec((_BB, KS), lambda i: (i, 0)),
            pl.BlockSpec((_BB, KS), lambda i: (i, 0)),
        ],
        out_specs=[
            pl.BlockSpec((1, 1), lambda i: (0, 0)),
            pl.BlockSpec((_BB, 1), lambda i: (i, 0)),
            pl.BlockSpec((_BB, 1), lambda i: (i, 0)),
        ],
        out_shape=[
            jax.ShapeDtypeStruct((1, 1), jnp.float32),
            jax.ShapeDtypeStruct((B, 1), jnp.float32),
            jax.ShapeDtypeStruct((B, 1), jnp.float32),
        ],
        interpret=interpret,
    )(g, student)


def kernel(teacher_logits_w, student_logits_s, t_idx, s_idx):
    t_idx = jnp.asarray(t_idx, jnp.int32)
    s_idx = jnp.asarray(s_idx, jnp.int32)
    g = _sc_gather(teacher_logits_w, t_idx, s_idx)
    loss2, c2, w2 = _tc_loss(g, student_logits_s)
    return (loss2[0, 0],
            jax.lax.stop_gradient(c2[:, 0]),
            jax.lax.stop_gradient(w2[:, 0]))


# speedup vs baseline: 4.5727x; 4.5727x over previous
import functools

import jax
import jax.numpy as jnp
from jax import lax
from jax.experimental import pallas as pl
from jax.experimental.pallas import tpu as pltpu
from jax.experimental.pallas import tpu_sc as plsc

TAU = 2.0
GAMMA = 0.7
EPS = 0.05
KS = 1000
KT = 100000
B = 1024

NUM_CORES = 2
NUM_SUBCORES = 16
NUM_WORKERS = NUM_CORES * NUM_SUBCORES
LANES = 16
NVEC = (KS + LANES - 1) // LANES

N_BANDS = B // 8
BANDS_PER_WORKER = N_BANDS // NUM_WORKERS
CHUNK_COLS = 6400
N_CHUNKS = 16
TAIL_COLS = 3968
J_PER_CHUNK = 64
OUT_W = 1024
PAD_CIDX = N_CHUNKS * CHUNK_COLS - CHUNK_COLS


def _sc_gather(teacher, t_idx, s_idx):
    mesh = plsc.VectorSubcoreMesh(
        core_axis_name="c", subcore_axis_name="s",
        num_cores=NUM_CORES, num_subcores=NUM_SUBCORES)

    @functools.partial(
        pl.kernel,
        out_type=jax.ShapeDtypeStruct((B, OUT_W), jnp.float32),
        mesh=mesh,
        scratch_types=[
            pltpu.VMEM((KS,), jnp.int32),
            pltpu.VMEM((KS,), jnp.int32),
            pltpu.VMEM((OUT_W,), jnp.int32),
            pltpu.VMEM((8, CHUNK_COLS), jnp.float32),
            pltpu.VMEM((8, CHUNK_COLS), jnp.float32),
            pltpu.VMEM((8, OUT_W), jnp.float32),
            pltpu.SemaphoreType.DMA,
            pltpu.SemaphoreType.DMA,
        ],
        compiler_params=pltpu.CompilerParams(needs_layout_passes=False),
    )
    def k(teacher_hbm, tidx_hbm, sidx_hbm, out_hbm, tv, sv, cv, buf0, buf1,
          obuf, sem0, sem1):
        wid = lax.axis_index("s") * NUM_CORES + lax.axis_index("c")
        pltpu.sync_copy(tidx_hbm, tv)
        pltpu.sync_copy(sidx_hbm, sv)

        pad = jnp.full((LANES,), PAD_CIDX, jnp.int32)
        cv[pl.ds(OUT_W - 2 * LANES, LANES)] = pad
        cv[pl.ds(OUT_W - LANES, LANES)] = pad

        def comp(j, carry):
            off = jnp.minimum(j * LANES, KS - LANES)
            plsc.store_scatter(
                cv, [sv[pl.ds(off, LANES)]], tv[pl.ds(off, LANES)])
            return carry
        lax.fori_loop(0, NVEC, comp, 0)

        base_band = wid * BANDS_PER_WORKER
        n_tasks = BANDS_PER_WORKER * N_CHUNKS

        def chunk_copy(t, buf, sem):
            band = t // N_CHUNKS
            c = t % N_CHUNKS
            b0 = (base_band + band) * 8
            c0 = c * CHUNK_COLS
            full = pltpu.make_async_copy(
                teacher_hbm.at[pl.ds(b0, 8), pl.ds(c0, CHUNK_COLS)], buf,
                sem)
            tail = pltpu.make_async_copy(
                teacher_hbm.at[pl.ds(b0, 8), pl.ds(c0, TAIL_COLS)],
                buf.at[:, pl.ds(0, TAIL_COLS)], sem)
            return c == N_CHUNKS - 1, full, tail

        def issue(t, buf, sem):
            is_tail, full, tail = chunk_copy(t, buf, sem)

            @pl.when(jnp.logical_not(is_tail))
            def _():
                full.start()

            @pl.when(is_tail)
            def _():
                tail.start()

        def wait(t, buf, sem):
            is_tail, full, tail = chunk_copy(t, buf, sem)

            @pl.when(jnp.logical_not(is_tail))
            def _():
                full.wait()

            @pl.when(is_tail)
            def _():
                tail.wait()

        def extract(t, buf):
            band = t // N_CHUNKS
            c = t % N_CHUNKS
            b0 = (base_band + band) * 8
            c0 = c * CHUNK_COLS
            j0 = c * J_PER_CHUNK
            for q in range(J_PER_CHUNK // LANES):
                lvec = cv[pl.ds(j0 + q * LANES, LANES)] - c0
                for r in range(8):
                    rvec = jnp.full((LANES,), r, jnp.int32)
                    obuf[r, pl.ds(j0 + q * LANES, LANES)] = plsc.load_gather(
                        buf, [rvec, lvec])

            @pl.when(c == N_CHUNKS - 1)
            def _():
                pltpu.sync_copy(obuf, out_hbm.at[pl.ds(b0, 8), :])

        issue(0, buf0, sem0)

        def pair(p, carry):
            t0 = 2 * p
            t1 = t0 + 1
            issue(t1, buf1, sem1)
            wait(t0, buf0, sem0)
            extract(t0, buf0)

            @pl.when(t1 + 1 < n_tasks)
            def _():
                issue(t1 + 1, buf0, sem0)
            wait(t1, buf1, sem1)
            extract(t1, buf1)
            return carry
        lax.fori_loop(0, n_tasks // 2, pair, 0)

    return k(teacher, t_idx, s_idx)


_BB = 128


def _tc_body(g_ref, s_ref, loss_ref, c_ref, w_ref):
    g = g_ref[...] * (1.0 / TAU)
    m = jnp.max(g, axis=1, keepdims=True)
    e = jnp.exp(g - m)
    p = e / jnp.sum(e, axis=1, keepdims=True)
    q = (1.0 - EPS) * p + (EPS / KS)

    st = s_ref[...] * (1.0 / TAU)
    ms = jnp.max(st, axis=1, keepdims=True)
    es = jnp.exp(st - ms)
    ps = es / jnp.sum(es, axis=1, keepdims=True)

    qc = jnp.maximum(q, 1e-12)
    psc = jnp.maximum(ps, 1e-12)
    kl = jnp.sum(qc * (jnp.log(qc) - jnp.log(psc)), axis=1)
    c = jnp.max(q, axis=1)
    w = jnp.clip((c - GAMMA) / (1.0 - GAMMA), 0.0, 1.0)
    c_ref[...] = c[:, None]
    w_ref[...] = w[:, None]

    part = jnp.sum(w * (TAU * TAU) * kl) * (1.0 / B)
    i = pl.program_id(0)

    @pl.when(i == 0)
    def _():
        loss_ref[...] = part[None, None]

    @pl.when(i != 0)
    def _():
        loss_ref[...] += part[None, None]


def _tc_loss(g, student, interpret=False):
    return pl.pallas_call(
        _tc_body,
        grid=(B // _BB,),
        in_specs=[
            pl.BlockSpec((_BB, KS), lambda i: (i, 0)),
            pl.BlockSpec((_BB, KS), lambda i: (i, 0)),
        ],
        out_specs=[
            pl.BlockSpec((1, 1), lambda i: (0, 0)),
            pl.BlockSpec((_BB, 1), lambda i: (i, 0)),
            pl.BlockSpec((_BB, 1), lambda i: (i, 0)),
        ],
        out_shape=[
            jax.ShapeDtypeStruct((1, 1), jnp.float32),
            jax.ShapeDtypeStruct((B, 1), jnp.float32),
            jax.ShapeDtypeStruct((B, 1), jnp.float32),
        ],
        interpret=interpret,
    )(g, student)


def kernel(teacher_logits_w, student_logits_s, t_idx, s_idx):
    t_idx = jnp.asarray(t_idx, jnp.int32)
    s_idx = jnp.asarray(s_idx, jnp.int32)
    g = _sc_gather(teacher_logits_w, t_idx, s_idx)[:, :KS]
    loss2, c2, w2 = _tc_loss(g, student_logits_s)
    return (loss2[0, 0],
            jax.lax.stop_gradient(c2[:, 0]),
            jax.lax.stop_gradient(w2[:, 0]))


# speedup vs baseline: 72.6662x; 15.8912x over previous
import functools

import jax
import jax.numpy as jnp
from jax import lax
from jax.experimental import pallas as pl
from jax.experimental.pallas import tpu as pltpu
from jax.experimental.pallas import tpu_sc as plsc

TAU = 2.0
GAMMA = 0.7
EPS = 0.05
KS = 1000
KT = 100000
B = 1024

NUM_CORES = 2
NUM_SUBCORES = 16
NUM_WORKERS = NUM_CORES * NUM_SUBCORES
LANES = 16
NVEC = (KS + LANES - 1) // LANES
OUT_ROWS = 1024
J_PER_WORKER = OUT_ROWS // NUM_WORKERS


def _sc_gather_t(teacher_t, t_idx, s_idx):
    mesh = plsc.VectorSubcoreMesh(
        core_axis_name="c", subcore_axis_name="s",
        num_cores=NUM_CORES, num_subcores=NUM_SUBCORES)

    @functools.partial(
        pl.kernel,
        out_type=jax.ShapeDtypeStruct((OUT_ROWS, B), jnp.float32),
        mesh=mesh,
        scratch_types=[
            pltpu.VMEM((KS,), jnp.int32),
            pltpu.VMEM((KS,), jnp.int32),
            pltpu.VMEM((OUT_ROWS,), jnp.int32),
            pltpu.VMEM((J_PER_WORKER, B), jnp.float32),
            pltpu.SemaphoreType.DMA,
        ],
        compiler_params=pltpu.CompilerParams(needs_layout_passes=False),
    )
    def k(teacher_hbm, tidx_hbm, sidx_hbm, out_hbm, tv, sv, cv, slab, sem):
        wid = lax.axis_index("s") * NUM_CORES + lax.axis_index("c")
        pltpu.sync_copy(tidx_hbm, tv)
        pltpu.sync_copy(sidx_hbm, sv)

        pad = jnp.zeros((LANES,), jnp.int32)
        cv[pl.ds(OUT_ROWS - 2 * LANES, LANES)] = pad
        cv[pl.ds(OUT_ROWS - LANES, LANES)] = pad

        def comp(j, carry):
            off = jnp.minimum(j * LANES, KS - LANES)
            plsc.store_scatter(
                cv, [sv[pl.ds(off, LANES)]], tv[pl.ds(off, LANES)])
            return carry
        lax.fori_loop(0, NVEC, comp, 0)

        j0 = wid * J_PER_WORKER

        pltpu.async_copy(
            teacher_hbm.at[cv.at[pl.ds(j0, J_PER_WORKER)]], slab,
            sem).wait()
        pltpu.sync_copy(slab, out_hbm.at[pl.ds(j0, J_PER_WORKER), :])

    return k(teacher_t, t_idx, s_idx)


_BBT = 256


def _tc_body(g_ref, s_ref, loss_ref, c_ref, w_ref):
    g = g_ref[...] * (1.0 / TAU)
    m = jnp.max(g, axis=0, keepdims=True)
    e = jnp.exp(g - m)
    p = e / jnp.sum(e, axis=0, keepdims=True)
    q = (1.0 - EPS) * p + (EPS / KS)

    st = s_ref[...] * (1.0 / TAU)
    ms = jnp.max(st, axis=0, keepdims=True)
    es = jnp.exp(st - ms)
    ps = es / jnp.sum(es, axis=0, keepdims=True)

    qc = jnp.maximum(q, 1e-12)
    psc = jnp.maximum(ps, 1e-12)
    kl = jnp.sum(qc * (jnp.log(qc) - jnp.log(psc)), axis=0, keepdims=True)
    c = jnp.max(q, axis=0, keepdims=True)
    w = jnp.clip((c - GAMMA) / (1.0 - GAMMA), 0.0, 1.0)
    c_ref[...] = c
    w_ref[...] = w

    part = jnp.sum(w * (TAU * TAU) * kl) * (1.0 / B)
    i = pl.program_id(0)

    @pl.when(i == 0)
    def _():
        loss_ref[...] = part[None, None]

    @pl.when(i != 0)
    def _():
        loss_ref[...] += part[None, None]


def _tc_loss(g_t, student_t, interpret=False):
    return pl.pallas_call(
        _tc_body,
        grid=(B // _BBT,),
        in_specs=[
            pl.BlockSpec((KS, _BBT), lambda i: (0, i)),
            pl.BlockSpec((KS, _BBT), lambda i: (0, i)),
        ],
        out_specs=[
            pl.BlockSpec((1, 1), lambda i: (0, 0)),
            pl.BlockSpec((1, _BBT), lambda i: (0, i)),
            pl.BlockSpec((1, _BBT), lambda i: (0, i)),
        ],
        out_shape=[
            jax.ShapeDtypeStruct((1, 1), jnp.float32),
            jax.ShapeDtypeStruct((1, B), jnp.float32),
            jax.ShapeDtypeStruct((1, B), jnp.float32),
        ],
        interpret=interpret,
    )(g_t, student_t)


def kernel(teacher_logits_w, student_logits_s, t_idx, s_idx):
    t_idx = jnp.asarray(t_idx, jnp.int32)
    s_idx = jnp.asarray(s_idx, jnp.int32)
    g_t = _sc_gather_t(teacher_logits_w.T, t_idx, s_idx)
    loss2, c2, w2 = _tc_loss(g_t, student_logits_s.T)
    return (loss2[0, 0],
            jax.lax.stop_gradient(c2[0]),
            jax.lax.stop_gradient(w2[0]))


# speedup vs baseline: 73.6267x; 1.0132x over previous
import functools

import jax
import jax.numpy as jnp
from jax import lax
from jax.experimental import pallas as pl
from jax.experimental.pallas import tpu as pltpu
from jax.experimental.pallas import tpu_sc as plsc

TAU = 2.0
GAMMA = 0.7
EPS = 0.05
KS = 1000
KT = 100000
B = 1024

NUM_CORES = 2
NUM_SUBCORES = 16
NUM_WORKERS = NUM_CORES * NUM_SUBCORES
LANES = 16
NVEC = (KS + LANES - 1) // LANES
OUT_ROWS = 1024
J_PER_WORKER = OUT_ROWS // NUM_WORKERS


def _sc_gather_t(teacher_t, t_idx, s_idx):
    mesh = plsc.VectorSubcoreMesh(
        core_axis_name="c", subcore_axis_name="s",
        num_cores=NUM_CORES, num_subcores=NUM_SUBCORES)

    @functools.partial(
        pl.kernel,
        out_type=jax.ShapeDtypeStruct((OUT_ROWS, B), jnp.float32),
        mesh=mesh,
        scratch_types=[
            pltpu.VMEM((KS,), jnp.int32),
            pltpu.VMEM((KS,), jnp.int32),
            pltpu.VMEM((OUT_ROWS,), jnp.int32),
            pltpu.VMEM((J_PER_WORKER, B), jnp.float32),
            pltpu.SemaphoreType.DMA,
        ],
        compiler_params=pltpu.CompilerParams(needs_layout_passes=False),
    )
    def k(teacher_hbm, tidx_hbm, sidx_hbm, out_hbm, tv, sv, cv, slab, sem):
        wid = lax.axis_index("s") * NUM_CORES + lax.axis_index("c")
        pltpu.sync_copy(tidx_hbm, tv)
        pltpu.sync_copy(sidx_hbm, sv)

        pad = jnp.zeros((LANES,), jnp.int32)
        cv[pl.ds(OUT_ROWS - 2 * LANES, LANES)] = pad
        cv[pl.ds(OUT_ROWS - LANES, LANES)] = pad

        def comp(j, carry):
            off = jnp.minimum(j * LANES, KS - LANES)
            plsc.store_scatter(
                cv, [sv[pl.ds(off, LANES)]], tv[pl.ds(off, LANES)])
            return carry
        lax.fori_loop(0, NVEC, comp, 0)

        j0 = wid * J_PER_WORKER

        pltpu.async_copy(
            teacher_hbm.at[cv.at[pl.ds(j0, J_PER_WORKER)]], slab,
            sem).wait()
        pltpu.sync_copy(slab, out_hbm.at[pl.ds(j0, J_PER_WORKER), :])

    return k(teacher_t, t_idx, s_idx)


_BBT = 256


def _tc_body(g_ref, s_ref, loss_ref, c_ref, w_ref):
    g = g_ref[...] * (1.0 / TAU)
    m = jnp.max(g, axis=0, keepdims=True)
    e = jnp.exp(g - m)
    p = e / jnp.sum(e, axis=0, keepdims=True)
    q = (1.0 - EPS) * p + (EPS / KS)

    st = s_ref[...] * (1.0 / TAU)
    ms = jnp.max(st, axis=0, keepdims=True)
    es = jnp.exp(st - ms)
    msl = ms + jnp.log(jnp.sum(es, axis=0, keepdims=True))

    qc = jnp.maximum(q, 1e-12)
    kl = (jnp.sum(qc * jnp.log(qc), axis=0, keepdims=True)
          - jnp.sum(qc * st, axis=0, keepdims=True)
          + msl * jnp.sum(qc, axis=0, keepdims=True))
    c = jnp.max(q, axis=0, keepdims=True)
    w = jnp.clip((c - GAMMA) / (1.0 - GAMMA), 0.0, 1.0)
    c_ref[...] = c
    w_ref[...] = w

    part = jnp.sum(w * (TAU * TAU) * kl) * (1.0 / B)
    i = pl.program_id(0)

    @pl.when(i == 0)
    def _():
        loss_ref[...] = part[None, None]

    @pl.when(i != 0)
    def _():
        loss_ref[...] += part[None, None]


def _tc_loss(g_t, student_t, interpret=False):
    return pl.pallas_call(
        _tc_body,
        grid=(B // _BBT,),
        in_specs=[
            pl.BlockSpec((KS, _BBT), lambda i: (0, i)),
            pl.BlockSpec((KS, _BBT), lambda i: (0, i)),
        ],
        out_specs=[
            pl.BlockSpec((1, 1), lambda i: (0, 0)),
            pl.BlockSpec((1, _BBT), lambda i: (0, i)),
            pl.BlockSpec((1, _BBT), lambda i: (0, i)),
        ],
        out_shape=[
            jax.ShapeDtypeStruct((1, 1), jnp.float32),
            jax.ShapeDtypeStruct((1, B), jnp.float32),
            jax.ShapeDtypeStruct((1, B), jnp.float32),
        ],
        interpret=interpret,
    )(g_t, student_t)


def kernel(teacher_logits_w, student_logits_s, t_idx, s_idx):
    t_idx = jnp.asarray(t_idx, jnp.int32)
    s_idx = jnp.asarray(s_idx, jnp.int32)
    g_t = _sc_gather_t(teacher_logits_w.T, t_idx, s_idx)
    loss2, c2, w2 = _tc_loss(g_t, student_logits_s.T)
    return (loss2[0, 0],
            jax.lax.stop_gradient(c2[0]),
            jax.lax.stop_gradient(w2[0]))
